# baseline (speedup 1.0000x reference)
import functools

import jax
import jax.numpy as jnp
from jax import lax
from jax.experimental import pallas as pl
from jax.experimental.pallas import tpu as pltpu
from jax.experimental.pallas import tpu_sc as plsc

_N = 10000
_E = 320000
_D_IN = 128
_D_H = 64

_NC = 2
_NS = 16
_NW = _NC * _NS
_CHUNK = 128

_NPAD = 10240
_RPT = _NPAD // _NS
_EPW = 10240
_E_PAD = _EPW * _NW
_NCHUNK = _EPW // _CHUNK
_DEG_W = 16
_NB = 4

_SC_PARAMS = pltpu.CompilerParams(use_tc_tiling_on_sc=False)


def _sc_degree(dst2d):
  mesh = plsc.VectorSubcoreMesh(core_axis_name="c", subcore_axis_name="s")

  @functools.partial(
      pl.kernel,
      out_type=jax.ShapeDtypeStruct((_NC * _N, _DEG_W), jnp.float32),
      mesh=mesh,
      scratch_types=[
          pltpu.VMEM((_NCHUNK, _CHUNK), jnp.int32),
          pltpu.VMEM((_CHUNK, _DEG_W), jnp.float32),
          pltpu.VMEM_SHARED((_NPAD, _DEG_W), jnp.float32),
          pltpu.SemaphoreType.DMA,
          pltpu.SemaphoreType.DMA,
      ],
      compiler_params=_SC_PARAMS,
  )
  def k(d_hbm, out_hbm, didx, buf, acc, isem, ssem):
    cid = lax.axis_index("c")
    sid = lax.axis_index("s")
    wid = sid * _NC + cid
    r0 = sid * _RPT

    idxc = pltpu.async_copy(
        d_hbm.at[pl.ds(wid * _NCHUNK, _NCHUNK)], didx, isem)

    def fill(i, val):
      buf[i] = jnp.full((_DEG_W,), val, jnp.float32)
      return val

    lax.fori_loop(0, _CHUNK, fill, 0.0)
    zcs = [
        pltpu.async_copy(buf, acc.at[pl.ds(r0 + j * _CHUNK, _CHUNK)], ssem)
        for j in range(_RPT // _CHUNK)
    ]
    tail = _RPT % _CHUNK
    if tail:
      zcs.append(pltpu.async_copy(
          buf.at[pl.ds(0, tail)],
          acc.at[pl.ds(r0 + (_RPT // _CHUNK) * _CHUNK, tail)], ssem))
    for c in zcs:
      c.wait()
    lax.fori_loop(0, _CHUNK, fill, 1.0)
    idxc.wait()
    plsc.subcore_barrier()

    def body(t, carry):
      cs = [
          pltpu.async_copy(buf, acc.at[didx.at[8 * t + b]], ssem, add=True)
          for b in range(8)
      ]
      for c in cs:
        c.wait()
      return carry

    lax.fori_loop(0, _NCHUNK // 8, body, 0)
    plsc.subcore_barrier()
    last = _N - (_NS - 1) * _RPT

    @pl.when(sid < _NS - 1)
    def _():
      pltpu.sync_copy(acc.at[pl.ds(r0, _RPT)],
                      out_hbm.at[pl.ds(cid * _N + r0, _RPT)])

    @pl.when(sid == _NS - 1)
    def _():
      pltpu.sync_copy(acc.at[pl.ds(r0, last)],
                      out_hbm.at[pl.ds(cid * _N + r0, last)])

  return k(dst2d)


def _sc_edge_pass(z, src2d, dst2d):
  mesh = plsc.VectorSubcoreMesh(core_axis_name="c", subcore_axis_name="s")

  @functools.partial(
      pl.kernel,
      out_type=jax.ShapeDtypeStruct((_NC * _N, _D_H), jnp.float32),
      mesh=mesh,
      scratch_types=[
          pltpu.VMEM((_NCHUNK, _CHUNK), jnp.int32),
          pltpu.VMEM((_NCHUNK, _CHUNK), jnp.int32),
          pltpu.VMEM((2 * _NB, _CHUNK, _D_H), jnp.float32),
          pltpu.VMEM_SHARED((_NPAD, _D_H), jnp.float32),
          pltpu.SemaphoreType.DMA,
          pltpu.SemaphoreType.DMA,
          pltpu.SemaphoreType.DMA,
          pltpu.SemaphoreType.DMA,
      ],
      compiler_params=_SC_PARAMS,
  )
  def k(z_hbm, s_hbm, d_hbm, out_hbm, sidx, didx, bufs, acc,
        gsA, gsB, ssA, ssB):
    cid = lax.axis_index("c")
    sid = lax.axis_index("s")
    wid = sid * _NC + cid
    r0 = sid * _RPT

    ic1 = pltpu.async_copy(
        s_hbm.at[pl.ds(wid * _NCHUNK, _NCHUNK)], sidx, gsB)
    ic2 = pltpu.async_copy(
        d_hbm.at[pl.ds(wid * _NCHUNK, _NCHUNK)], didx, gsB)

    zb = bufs.at[0]

    def zstore(i, carry):
      r = i // (_D_H // 16)
      c = (i % (_D_H // 16)) * 16
      zb[r, pl.ds(c, 16)] = jnp.zeros((16,), jnp.float32)
      return carry

    lax.fori_loop(0, _CHUNK * (_D_H // 16), zstore, 0)
    zcs = [
        pltpu.async_copy(zb, acc.at[pl.ds(r0 + j * _CHUNK, _CHUNK)], ssA)
        for j in range(_RPT // _CHUNK)
    ]
    tail = _RPT % _CHUNK
    if tail:
      zcs.append(pltpu.async_copy(
          zb.at[pl.ds(0, tail)],
          acc.at[pl.ds(r0 + (_RPT // _CHUNK) * _CHUNK, tail)], ssA))
    for c in zcs:
      c.wait()
    ic1.wait()
    ic2.wait()
    plsc.subcore_barrier()

    def gfire(j, b, sem):
      pltpu.async_copy(z_hbm.at[sidx.at[j]], bufs.at[b], sem)

    def gwait(j, b, sem):
      pltpu.make_async_copy(z_hbm.at[sidx.at[j]], bufs.at[b], sem).wait()

    def sfire(j, b, sem):
      pltpu.async_copy(bufs.at[b], acc.at[didx.at[j]], sem, add=True)

    def swait(j, b, sem):
      pltpu.make_async_copy(bufs.at[b], acc.at[didx.at[j]], sem).wait()

    for b in range(_NB):
      gfire(b, b, gsA)

    def body(t, carry):
      jA = t * 2 * _NB
      jB = jA + _NB
      for b in range(_NB):
        gwait(jA + b, b, gsA)
      for b in range(_NB):
        gfire(jB + b, _NB + b, gsB)
      for b in range(_NB):
        sfire(jA + b, b, ssA)
      for b in range(_NB):
        swait(jA + b, b, ssA)

      @pl.when(t < _NCHUNK // (2 * _NB) - 1)
      def _():
        for b in range(_NB):
          gfire(jA + 2 * _NB + b, b, gsA)

      for b in range(_NB):
        gwait(jB + b, _NB + b, gsB)
      for b in range(_NB):
        sfire(jB + b, _NB + b, ssB)
      for b in range(_NB):
        swait(jB + b, _NB + b, ssB)
      return carry

    lax.fori_loop(0, _NCHUNK // (2 * _NB), body, 0)
    plsc.subcore_barrier()
    last = _N - (_NS - 1) * _RPT

    @pl.when(sid < _NS - 1)
    def _():
      pltpu.sync_copy(acc.at[pl.ds(r0, _RPT)],
                      out_hbm.at[pl.ds(cid * _N + r0, _RPT)])

    @pl.when(sid == _NS - 1)
    def _():
      pltpu.sync_copy(acc.at[pl.ds(r0, last)],
                      out_hbm.at[pl.ds(cid * _N + r0, last)])

  return k(z, src2d, dst2d)


_BR = 1000


def _tc_layer1(x, W1, deg0, deg1):
  def body(x_ref, w_ref, d0_ref, d1_ref, z_ref):
    deg = d0_ref[:, 0:1] + d1_ref[:, 0:1] + 1.0
    dinv = 1.0 / jnp.sqrt(deg)
    xw = jnp.dot(x_ref[...], w_ref[...], preferred_element_type=jnp.float32)
    z_ref[...] = dinv * xw

  return pl.pallas_call(
      body,
      grid=(_N // _BR,),
      in_specs=[
          pl.BlockSpec((_BR, _D_IN), lambda i: (i, 0)),
          pl.BlockSpec((_D_IN, _D_H), lambda i: (0, 0)),
          pl.BlockSpec((_BR, _DEG_W), lambda i: (i, 0)),
          pl.BlockSpec((_BR, _DEG_W), lambda i: (i + _N // _BR, 0)),
      ],
      out_specs=pl.BlockSpec((_BR, _D_H), lambda i: (i, 0)),
      out_shape=jax.ShapeDtypeStruct((_N, _D_H), jnp.float32),
  )(x, W1, deg0, deg1)


def _tc_mid(a0, a1, z1, deg0, deg1, b1, W2):
  def body(a0_ref, a1_ref, z_ref, d0_ref, d1_ref, b_ref, w_ref, z2_ref):
    dv = 1.0 / jnp.sqrt(d0_ref[:, 0:1] + d1_ref[:, 0:1] + 1.0)
    h = jnp.maximum(
        dv * (a0_ref[...] + a1_ref[...] + z_ref[...]) + b_ref[...], 0.0)
    z2_ref[...] = dv * jnp.dot(h, w_ref[...],
                               preferred_element_type=jnp.float32)

  return pl.pallas_call(
      body,
      grid=(_N // _BR,),
      in_specs=[
          pl.BlockSpec((_BR, _D_H), lambda i: (i, 0)),
          pl.BlockSpec((_BR, _D_H), lambda i: (i + _N // _BR, 0)),
          pl.BlockSpec((_BR, _D_H), lambda i: (i, 0)),
          pl.BlockSpec((_BR, _DEG_W), lambda i: (i, 0)),
          pl.BlockSpec((_BR, _DEG_W), lambda i: (i + _N // _BR, 0)),
          pl.BlockSpec((1, _D_H), lambda i: (0, 0)),
          pl.BlockSpec((_D_H, _D_H), lambda i: (0, 0)),
      ],
      out_specs=pl.BlockSpec((_BR, _D_H), lambda i: (i, 0)),
      out_shape=jax.ShapeDtypeStruct((_N, _D_H), jnp.float32),
  )(a0, a1, z1, deg0, deg1, b1, W2)


def _tc_final(a0, a1, z2, deg0, deg1, b2):
  def body(a0_ref, a1_ref, z_ref, d0_ref, d1_ref, b_ref, o_ref):
    dv = 1.0 / jnp.sqrt(d0_ref[:, 0:1] + d1_ref[:, 0:1] + 1.0)
    o_ref[...] = dv * (a0_ref[...] + a1_ref[...] + z_ref[...]) + b_ref[...]

  return pl.pallas_call(
      body,
      grid=(_N // _BR,),
      in_specs=[
          pl.BlockSpec((_BR, _D_H), lambda i: (i, 0)),
          pl.BlockSpec((_BR, _D_H), lambda i: (i + _N // _BR, 0)),
          pl.BlockSpec((_BR, _D_H), lambda i: (i, 0)),
          pl.BlockSpec((_BR, _DEG_W), lambda i: (i, 0)),
          pl.BlockSpec((_BR, _DEG_W), lambda i: (i + _N // _BR, 0)),
          pl.BlockSpec((1, _D_H), lambda i: (0, 0)),
      ],
      out_specs=pl.BlockSpec((_BR, _D_H), lambda i: (i, 0)),
      out_shape=jax.ShapeDtypeStruct((_N, _D_H), jnp.float32),
  )(a0, a1, z2, deg0, deg1, b2)


def kernel(x, edge_index, W1, b1, W2, b2):
  src = edge_index[0]
  dst = edge_index[1]

  pad = _E_PAD - _E
  pad_src = jnp.arange(pad, dtype=jnp.int32)
  pad_dst = _N + jnp.broadcast_to(
      jnp.arange(_NPAD - _N, dtype=jnp.int32),
      (pad // (_NPAD - _N), _NPAD - _N)).reshape(pad)
  srcp = jnp.concatenate([src, pad_src]).reshape(_E_PAD // _CHUNK, _CHUNK)
  dstp = jnp.concatenate([dst, pad_dst]).reshape(_E_PAD // _CHUNK, _CHUNK)

  degp = _sc_degree(dstp)
  z1 = _tc_layer1(x, W1, degp, degp)

  acc1 = _sc_edge_pass(z1, srcp, dstp)
  z2 = _tc_mid(acc1, acc1, z1, degp, degp, b1.reshape(1, _D_H), W2)

  acc2 = _sc_edge_pass(z2, srcp, dstp)
  return _tc_final(acc2, acc2, z2, degp, degp, b2.reshape(1, _D_H))


# speedup vs baseline: 1.1311x; 1.1311x over previous
import functools

import jax
import jax.numpy as jnp
from jax import lax
from jax.experimental import pallas as pl
from jax.experimental.pallas import tpu as pltpu
from jax.experimental.pallas import tpu_sc as plsc

_N = 10000
_E = 320000
_D_IN = 128
_D_H = 64

_NC = 2
_NS = 16
_NW = _NC * _NS
_CHUNK = 128

_NPAD = 10240
_RPT = _NPAD // _NS
_EPW = 10240
_E_PAD = _EPW * _NW
_NCHUNK = _EPW // _CHUNK
_DEG_W = 16
_NB = 4

_SC_PARAMS = pltpu.CompilerParams(use_tc_tiling_on_sc=False)


def _sc_degree(ei):
  mesh = plsc.VectorSubcoreMesh(core_axis_name="c", subcore_axis_name="s")

  @functools.partial(
      pl.kernel,
      out_type=jax.ShapeDtypeStruct((_NC * _N, _DEG_W), jnp.float32),
      mesh=mesh,
      scratch_types=[
          pltpu.VMEM((_NCHUNK, _CHUNK), jnp.int32),
          pltpu.VMEM((_CHUNK, _DEG_W), jnp.float32),
          pltpu.VMEM_SHARED((_NPAD, _DEG_W), jnp.float32),
          pltpu.SemaphoreType.DMA,
          pltpu.SemaphoreType.DMA,
      ],
      compiler_params=_SC_PARAMS,
  )
  def k(ei_hbm, out_hbm, didx, buf, acc, isem, ssem):
    cid = lax.axis_index("c")
    sid = lax.axis_index("s")
    wid = sid * _NC + cid
    r0 = sid * _RPT

    idxc = pltpu.async_copy(
        ei_hbm.at[1, pl.ds(wid * _NCHUNK, _NCHUNK)], didx, isem)

    def fill(i, val):
      buf[i] = jnp.full((_DEG_W,), val, jnp.float32)
      return val

    lax.fori_loop(0, _CHUNK, fill, 0.0)
    zcs = [
        pltpu.async_copy(buf, acc.at[pl.ds(r0 + j * _CHUNK, _CHUNK)], ssem)
        for j in range(_RPT // _CHUNK)
    ]
    tail = _RPT % _CHUNK
    if tail:
      zcs.append(pltpu.async_copy(
          buf.at[pl.ds(0, tail)],
          acc.at[pl.ds(r0 + (_RPT // _CHUNK) * _CHUNK, tail)], ssem))
    for c in zcs:
      c.wait()
    lax.fori_loop(0, _CHUNK, fill, 1.0)
    idxc.wait()
    plsc.subcore_barrier()

    def body(t, carry):
      cs = [
          pltpu.async_copy(buf, acc.at[didx.at[8 * t + b]], ssem, add=True)
          for b in range(8)
      ]
      for c in cs:
        c.wait()
      return carry

    lax.fori_loop(0, _NCHUNK // 8, body, 0)
    plsc.subcore_barrier()
    last = _N - (_NS - 1) * _RPT

    @pl.when(sid < _NS - 1)
    def _():
      pltpu.sync_copy(acc.at[pl.ds(r0, _RPT)],
                      out_hbm.at[pl.ds(cid * _N + r0, _RPT)])

    @pl.when(sid == _NS - 1)
    def _():
      pltpu.sync_copy(acc.at[pl.ds(r0, last)],
                      out_hbm.at[pl.ds(cid * _N + r0, last)])

  return k(ei)


def _sc_edge_pass(z, ei):
  mesh = plsc.VectorSubcoreMesh(core_axis_name="c", subcore_axis_name="s")

  @functools.partial(
      pl.kernel,
      out_type=jax.ShapeDtypeStruct((_NC * _N, 128), jnp.float32),
      mesh=mesh,
      scratch_types=[
          pltpu.VMEM((_NCHUNK, _CHUNK), jnp.int32),
          pltpu.VMEM((_NCHUNK, _CHUNK), jnp.int32),
          pltpu.VMEM((2 * _NB, _CHUNK, _D_H), jnp.float32),
          pltpu.VMEM_SHARED((_NPAD, _D_H), jnp.float32),
          pltpu.SemaphoreType.DMA,
          pltpu.SemaphoreType.DMA,
          pltpu.SemaphoreType.DMA,
          pltpu.SemaphoreType.DMA,
      ],
      compiler_params=_SC_PARAMS,
  )
  def k(z_hbm, ei_hbm, out_hbm, sidx, didx, bufs, acc,
        gsA, gsB, ssA, ssB):
    cid = lax.axis_index("c")
    sid = lax.axis_index("s")
    wid = sid * _NC + cid
    r0 = sid * _RPT

    ic1 = pltpu.async_copy(
        ei_hbm.at[0, pl.ds(wid * _NCHUNK, _NCHUNK)], sidx, gsB)
    ic2 = pltpu.async_copy(
        ei_hbm.at[1, pl.ds(wid * _NCHUNK, _NCHUNK)], didx, gsB)

    zb = bufs.at[0]

    def zstore(i, carry):
      r = i // (_D_H // 16)
      c = (i % (_D_H // 16)) * 16
      zb[r, pl.ds(c, 16)] = jnp.zeros((16,), jnp.float32)
      return carry

    lax.fori_loop(0, _CHUNK * (_D_H // 16), zstore, 0)
    zcs = [
        pltpu.async_copy(zb, acc.at[pl.ds(r0 + j * _CHUNK, _CHUNK)], ssA)
        for j in range(_RPT // _CHUNK)
    ]
    tail = _RPT % _CHUNK
    if tail:
      zcs.append(pltpu.async_copy(
          zb.at[pl.ds(0, tail)],
          acc.at[pl.ds(r0 + (_RPT // _CHUNK) * _CHUNK, tail)], ssA))
    for c in zcs:
      c.wait()
    ic1.wait()
    ic2.wait()
    plsc.subcore_barrier()

    def gfire(j, b, sem):
      pltpu.async_copy(z_hbm.at[sidx.at[j]], bufs.at[b], sem)

    def gwait(j, b, sem):
      pltpu.make_async_copy(z_hbm.at[sidx.at[j]], bufs.at[b], sem).wait()

    def sfire(j, b, sem):
      pltpu.async_copy(bufs.at[b], acc.at[didx.at[j]], sem, add=True)

    def swait(j, b, sem):
      pltpu.make_async_copy(bufs.at[b], acc.at[didx.at[j]], sem).wait()

    for b in range(_NB):
      gfire(b, b, gsA)

    def body(t, carry):
      jA = t * 2 * _NB
      jB = jA + _NB
      for b in range(_NB):
        gwait(jA + b, b, gsA)
      for b in range(_NB):
        gfire(jB + b, _NB + b, gsB)
      for b in range(_NB):
        sfire(jA + b, b, ssA)
      for b in range(_NB):
        swait(jA + b, b, ssA)

      @pl.when(t < _NCHUNK // (2 * _NB) - 1)
      def _():
        for b in range(_NB):
          gfire(jA + 2 * _NB + b, b, gsA)

      for b in range(_NB):
        gwait(jB + b, _NB + b, gsB)
      for b in range(_NB):
        sfire(jB + b, _NB + b, ssB)
      for b in range(_NB):
        swait(jB + b, _NB + b, ssB)
      return carry

    lax.fori_loop(0, _NCHUNK // (2 * _NB), body, 0)
    plsc.subcore_barrier()
    last = _N - (_NS - 1) * _RPT

    @pl.when(sid < _NS - 1)
    def _():
      pltpu.sync_copy(acc.at[pl.ds(r0, _RPT)],
                      out_hbm.at[pl.ds(cid * _N + r0, _RPT), pl.ds(0, _D_H)])

    @pl.when(sid == _NS - 1)
    def _():
      pltpu.sync_copy(acc.at[pl.ds(r0, last)],
                      out_hbm.at[pl.ds(cid * _N + r0, last), pl.ds(0, _D_H)])

  return k(z, ei)


_BR = 1000


def _tc_layer1(x, W1, deg0, deg1):
  def body(x_ref, w_ref, d0_ref, d1_ref, z_ref):
    deg = d0_ref[:, 0:1] + d1_ref[:, 0:1] + 1.0
    dinv = 1.0 / jnp.sqrt(deg)
    xw = jnp.dot(x_ref[...], w_ref[...], preferred_element_type=jnp.float32)
    z_ref[...] = dinv * xw

  return pl.pallas_call(
      body,
      grid=(_N // _BR,),
      in_specs=[
          pl.BlockSpec((_BR, _D_IN), lambda i: (i, 0)),
          pl.BlockSpec((_D_IN, _D_H), lambda i: (0, 0)),
          pl.BlockSpec((_BR, _DEG_W), lambda i: (i, 0)),
          pl.BlockSpec((_BR, _DEG_W), lambda i: (i + _N // _BR, 0)),
      ],
      out_specs=pl.BlockSpec((_BR, _D_H), lambda i: (i, 0)),
      out_shape=jax.ShapeDtypeStruct((_N, _D_H), jnp.float32),
  )(x, W1, deg0, deg1)


def _tc_mid(a0, a1, z1, deg0, deg1, b1, W2):
  def body(a0_ref, a1_ref, z_ref, d0_ref, d1_ref, b_ref, w_ref, z2_ref):
    dv = 1.0 / jnp.sqrt(d0_ref[:, 0:1] + d1_ref[:, 0:1] + 1.0)
    asum = a0_ref[:, :_D_H] + a1_ref[:, :_D_H]
    h = jnp.maximum(dv * (asum + z_ref[...]) + b_ref[...], 0.0)
    z2_ref[...] = dv * jnp.dot(h, w_ref[...],
                               preferred_element_type=jnp.float32)

  return pl.pallas_call(
      body,
      grid=(_N // _BR,),
      in_specs=[
          pl.BlockSpec((_BR, 128), lambda i: (i, 0)),
          pl.BlockSpec((_BR, 128), lambda i: (i + _N // _BR, 0)),
          pl.BlockSpec((_BR, _D_H), lambda i: (i, 0)),
          pl.BlockSpec((_BR, _DEG_W), lambda i: (i, 0)),
          pl.BlockSpec((_BR, _DEG_W), lambda i: (i + _N // _BR, 0)),
          pl.BlockSpec((1, _D_H), lambda i: (0, 0)),
          pl.BlockSpec((_D_H, _D_H), lambda i: (0, 0)),
      ],
      out_specs=pl.BlockSpec((_BR, _D_H), lambda i: (i, 0)),
      out_shape=jax.ShapeDtypeStruct((_N, _D_H), jnp.float32),
  )(a0, a1, z1, deg0, deg1, b1, W2)


def _tc_final(a0, a1, z2, deg0, deg1, b2):
  def body(a0_ref, a1_ref, z_ref, d0_ref, d1_ref, b_ref, o_ref):
    dv = 1.0 / jnp.sqrt(d0_ref[:, 0:1] + d1_ref[:, 0:1] + 1.0)
    asum = a0_ref[:, :_D_H] + a1_ref[:, :_D_H]
    o_ref[...] = dv * (asum + z_ref[...]) + b_ref[...]

  return pl.pallas_call(
      body,
      grid=(_N // _BR,),
      in_specs=[
          pl.BlockSpec((_BR, 128), lambda i: (i, 0)),
          pl.BlockSpec((_BR, 128), lambda i: (i + _N // _BR, 0)),
          pl.BlockSpec((_BR, _D_H), lambda i: (i, 0)),
          pl.BlockSpec((_BR, _DEG_W), lambda i: (i, 0)),
          pl.BlockSpec((_BR, _DEG_W), lambda i: (i + _N // _BR, 0)),
          pl.BlockSpec((1, _D_H), lambda i: (0, 0)),
      ],
      out_specs=pl.BlockSpec((_BR, _D_H), lambda i: (i, 0)),
      out_shape=jax.ShapeDtypeStruct((_N, _D_H), jnp.float32),
  )(a0, a1, z2, deg0, deg1, b2)


def kernel(x, edge_index, W1, b1, W2, b2):
  pad = _E_PAD - _E
  pad_src = jnp.arange(pad, dtype=jnp.int32)
  pad_dst = _N + jnp.broadcast_to(
      jnp.arange(_NPAD - _N, dtype=jnp.int32),
      (pad // (_NPAD - _N), _NPAD - _N)).reshape(pad)
  ei = jnp.concatenate(
      [edge_index, jnp.stack([pad_src, pad_dst])],
      axis=1).reshape(2, _E_PAD // _CHUNK, _CHUNK)

  degp = _sc_degree(ei)
  z1 = _tc_layer1(x, W1, degp, degp)

  acc1 = _sc_edge_pass(z1, ei)
  z2 = _tc_mid(acc1, acc1, z1, degp, degp, b1.reshape(1, _D_H), W2)

  acc2 = _sc_edge_pass(z2, ei)
  return _tc_final(acc2, acc2, z2, degp, degp, b2.reshape(1, _D_H))
